# baseline (speedup 1.0000x reference)
import functools

import jax
import jax.numpy as jnp
from jax import lax
from jax.experimental import pallas as pl
from jax.experimental.pallas import tpu as pltpu
from jax.experimental.pallas import tpu_sc as plsc

B = 16384
D = 64
W = 128
G = 256
NC, NS = 2, 16
NW = NC * NS
BPW = B // NW
BK = 1024


def _repack_body(t_ref, out_ref):
    y = t_ref[...].T
    out_ref[...] = jnp.concatenate([y[:W], y[W:]], axis=1)


def _repack(tab_t):
    n = tab_t.shape[1]
    ng = (n + G - 1) // G
    return pl.pallas_call(
        _repack_body,
        grid=(ng,),
        in_specs=[pl.BlockSpec((D, G), lambda j: (0, j))],
        out_specs=pl.BlockSpec((W, W), lambda j: (j, 0)),
        out_shape=jax.ShapeDtypeStruct((ng * W, W), jnp.float32),
    )(tab_t)


def _gather_body(user_idx, item_idx, pos_idx, neg_idx,
                 user_tab, item_tab, tag_tab,
                 u_out, i_out, p_out, n_out,
                 idx_v, rows_v, sem):
    wid = lax.axis_index("s") * NC + lax.axis_index("c")
    base = wid * BPW
    jobs = ((user_idx, user_tab, u_out),
            (item_idx, item_tab, i_out),
            (pos_idx, tag_tab, p_out),
            (neg_idx, tag_tab, n_out))
    for idx_hbm, tab, out in jobs:
        pltpu.sync_copy(idx_hbm.at[pl.ds(base, BPW)], idx_v)
        for k in range(BPW // 16):
            sl = pl.ds(k * 16, 16)
            v = idx_v[sl]
            idx_v[sl] = lax.bitwise_or(
                lax.shift_left(lax.shift_right_logical(v, 8), 7),
                lax.bitwise_and(v, 127))
        pltpu.async_copy(tab.at[idx_v], rows_v, sem).wait()
        pltpu.sync_copy(rows_v, out.at[pl.ds(base, BPW)])


@jax.jit
def _gather(user, item, pos_tag, neg_tag, user_tab2, item_tab2, tag_tab2):
    mesh = plsc.VectorSubcoreMesh(core_axis_name="c", subcore_axis_name="s",
                                  num_cores=NC, num_subcores=NS)
    emb = jax.ShapeDtypeStruct((B, W), jnp.float32)
    run = pl.kernel(
        _gather_body,
        out_type=(emb, emb, emb, emb),
        mesh=mesh,
        scratch_types=[
            pltpu.VMEM((BPW,), jnp.int32),
            pltpu.VMEM((BPW, W), jnp.float32),
            pltpu.SemaphoreType.DMA,
        ],
    )
    return run(user, item, pos_tag, neg_tag, user_tab2, item_tab2, tag_tab2)


def _sel(x2, idx):
    par = lax.bitwise_and(lax.shift_right_logical(idx, 7), 1) == 1
    return jnp.where(par, x2[:, D:], x2[:, :D])


def _score_body(u_ref, i_ref, p_ref, n_ref, ui_ref, ii_ref, pi_ref, ni_ref,
                e_ref, c_ref, out_ref):
    u_emb = _sel(u_ref[...], ui_ref[...])
    i_emb = _sel(i_ref[...], ii_ref[...])
    u_bf = u_emb.astype(jnp.bfloat16)
    i_bf = i_emb.astype(jnp.bfloat16)
    u_exp = jnp.dot(u_bf, e_ref[...],
                    preferred_element_type=jnp.float32).astype(jnp.bfloat16)
    i_tiled = pltpu.repeat(i_bf, D, axis=1)
    p_outer = u_exp * i_tiled
    w = jnp.dot(p_outer, c_ref[...], preferred_element_type=jnp.float32)
    d = _sel(p_ref[...], pi_ref[...]) - _sel(n_ref[...], ni_ref[...])
    s = jnp.sum(w * d, axis=1)
    ls = jnp.minimum(s, 0.0) - jnp.log(1.0 + jnp.exp(-jnp.abs(s)))
    part = jnp.sum(ls)

    @pl.when(pl.program_id(0) == 0)
    def _init():
        out_ref[0, 0] = 0.0

    out_ref[0, 0] += part

    @pl.when(pl.program_id(0) == pl.num_programs(0) - 1)
    def _fin():
        out_ref[0, 0] = out_ref[0, 0] * (-1.0 / B)


@jax.jit
def _score(u2, i2, p2, n2, ui, ii, pi, ni, core_tensor):
    e_mat = jnp.repeat(jnp.eye(D, dtype=jnp.bfloat16), D, axis=1)
    c_mat = core_tensor.reshape(D * D, D).astype(jnp.bfloat16)
    row = pl.BlockSpec((BK, W), lambda i: (i, 0))
    col = pl.BlockSpec((BK, 1), lambda i: (i, 0))
    loss = pl.pallas_call(
        _score_body,
        grid=(B // BK,),
        in_specs=[
            row, row, row, row,
            col, col, col, col,
            pl.BlockSpec((D, D * D), lambda i: (0, 0)),
            pl.BlockSpec((D * D, D), lambda i: (0, 0)),
        ],
        out_specs=pl.BlockSpec((1, 1), lambda i: (0, 0),
                               memory_space=pltpu.SMEM),
        out_shape=jax.ShapeDtypeStruct((1, 1), jnp.float32),
    )(u2, i2, p2, n2, ui, ii, pi, ni, e_mat, c_mat)
    return loss[0, 0]


def kernel(user, item, pos_tag, neg_tag, user_table, item_table,
           good_tag_table, core_tensor):
    ut2 = _repack(user_table.T)
    it2 = _repack(item_table.T)
    tt2 = _repack(good_tag_table.T)
    u2, i2, p2, n2 = _gather(user, item, pos_tag, neg_tag, ut2, it2, tt2)
    col = lambda idx: idx.reshape(B, 1)
    return _score(u2, i2, p2, n2, col(user), col(item), col(pos_tag),
                  col(neg_tag), core_tensor)


# speedup vs baseline: 6.3123x; 6.3123x over previous
import functools

import jax
import jax.numpy as jnp
from jax import lax
from jax.experimental import pallas as pl
from jax.experimental.pallas import tpu as pltpu
from jax.experimental.pallas import tpu_sc as plsc

B = 16384
D = 64
W = 128
G = 16384
H = G // 2
NC, NS = 2, 16
NW = NC * NS
BPW = B // NW
BK = 1024


def _repack_body(t_ref, out_ref):
    y = t_ref[...].T
    out_ref[...] = jnp.concatenate([y[:H], y[H:]], axis=1)


def _repack(tab_t):
    n = tab_t.shape[1]
    ng = (n + G - 1) // G
    return pl.pallas_call(
        _repack_body,
        grid=(ng,),
        in_specs=[pl.BlockSpec((D, G), lambda j: (0, j))],
        out_specs=pl.BlockSpec((H, W), lambda j: (j, 0)),
        out_shape=jax.ShapeDtypeStruct((ng * H, W), jnp.float32),
    )(tab_t)


def _gather_body(user_idx, item_idx, pos_idx, neg_idx,
                 user_tab, item_tab, tag_tab,
                 u_out, i_out, p_out, n_out,
                 idx_v, rows_v, sem):
    wid = lax.axis_index("s") * NC + lax.axis_index("c")
    base = wid * BPW
    jobs = ((user_idx, user_tab, u_out),
            (item_idx, item_tab, i_out),
            (pos_idx, tag_tab, p_out),
            (neg_idx, tag_tab, n_out))
    for idx_hbm, tab, out in jobs:
        pltpu.sync_copy(idx_hbm.at[pl.ds(base, BPW)], idx_v)
        for k in range(BPW // 16):
            sl = pl.ds(k * 16, 16)
            v = idx_v[sl]
            idx_v[sl] = lax.bitwise_or(
                lax.shift_left(lax.shift_right_logical(v, 14), 13),
                lax.bitwise_and(v, H - 1))
        pltpu.async_copy(tab.at[idx_v], rows_v, sem).wait()
        pltpu.sync_copy(rows_v, out.at[pl.ds(base, BPW)])


@jax.jit
def _gather(user, item, pos_tag, neg_tag, user_tab2, item_tab2, tag_tab2):
    mesh = plsc.VectorSubcoreMesh(core_axis_name="c", subcore_axis_name="s",
                                  num_cores=NC, num_subcores=NS)
    emb = jax.ShapeDtypeStruct((B, W), jnp.float32)
    run = pl.kernel(
        _gather_body,
        out_type=(emb, emb, emb, emb),
        mesh=mesh,
        scratch_types=[
            pltpu.VMEM((BPW,), jnp.int32),
            pltpu.VMEM((BPW, W), jnp.float32),
            pltpu.SemaphoreType.DMA,
        ],
    )
    return run(user, item, pos_tag, neg_tag, user_tab2, item_tab2, tag_tab2)


def _sel(x2, idx):
    par = lax.bitwise_and(lax.shift_right_logical(idx, 13), 1) == 1
    return jnp.where(par, x2[:, D:], x2[:, :D])


def _score_body(u_ref, i_ref, p_ref, n_ref, ui_ref, ii_ref, pi_ref, ni_ref,
                e_ref, c_ref, out_ref):
    u_emb = _sel(u_ref[...], ui_ref[...])
    i_emb = _sel(i_ref[...], ii_ref[...])
    u_bf = u_emb.astype(jnp.bfloat16)
    i_bf = i_emb.astype(jnp.bfloat16)
    u_exp = jnp.dot(u_bf, e_ref[...],
                    preferred_element_type=jnp.float32).astype(jnp.bfloat16)
    i_tiled = pltpu.repeat(i_bf, D, axis=1)
    p_outer = u_exp * i_tiled
    w = jnp.dot(p_outer, c_ref[...], preferred_element_type=jnp.float32)
    d = _sel(p_ref[...], pi_ref[...]) - _sel(n_ref[...], ni_ref[...])
    s = jnp.sum(w * d, axis=1)
    ls = jnp.minimum(s, 0.0) - jnp.log(1.0 + jnp.exp(-jnp.abs(s)))
    part = jnp.sum(ls)

    @pl.when(pl.program_id(0) == 0)
    def _init():
        out_ref[0, 0] = 0.0

    out_ref[0, 0] += part

    @pl.when(pl.program_id(0) == pl.num_programs(0) - 1)
    def _fin():
        out_ref[0, 0] = out_ref[0, 0] * (-1.0 / B)


@jax.jit
def _score(u2, i2, p2, n2, ui, ii, pi, ni, core_tensor):
    e_mat = jnp.repeat(jnp.eye(D, dtype=jnp.bfloat16), D, axis=1)
    c_mat = core_tensor.reshape(D * D, D).astype(jnp.bfloat16)
    row = pl.BlockSpec((BK, W), lambda i: (i, 0))
    col = pl.BlockSpec((BK, 1), lambda i: (i, 0))
    loss = pl.pallas_call(
        _score_body,
        grid=(B // BK,),
        in_specs=[
            row, row, row, row,
            col, col, col, col,
            pl.BlockSpec((D, D * D), lambda i: (0, 0)),
            pl.BlockSpec((D * D, D), lambda i: (0, 0)),
        ],
        out_specs=pl.BlockSpec((1, 1), lambda i: (0, 0),
                               memory_space=pltpu.SMEM),
        out_shape=jax.ShapeDtypeStruct((1, 1), jnp.float32),
    )(u2, i2, p2, n2, ui, ii, pi, ni, e_mat, c_mat)
    return loss[0, 0]


def kernel(user, item, pos_tag, neg_tag, user_table, item_table,
           good_tag_table, core_tensor):
    ut2 = _repack(user_table.T)
    it2 = _repack(item_table.T)
    tt2 = _repack(good_tag_table.T)
    u2, i2, p2, n2 = _gather(user, item, pos_tag, neg_tag, ut2, it2, tt2)
    col = lambda idx: idx.reshape(B, 1)
    return _score(u2, i2, p2, n2, col(user), col(item), col(pos_tag),
                  col(neg_tag), core_tensor)


# speedup vs baseline: 6.4032x; 1.0144x over previous
import functools

import jax
import jax.numpy as jnp
from jax import lax
from jax.experimental import pallas as pl
from jax.experimental.pallas import tpu as pltpu
from jax.experimental.pallas import tpu_sc as plsc

B = 16384
D = 64
W = 128
G = 16384
H = G // 2
NC, NS = 2, 16
NW = NC * NS
BPW = B // NW
BK = 1024


def _repack_body(t_ref, out_ref):
    y = t_ref[...].T
    out_ref[...] = jnp.concatenate([y[:H], y[H:]], axis=1)


def _repack(tab_t):
    n = tab_t.shape[1]
    ng = (n + G - 1) // G
    return pl.pallas_call(
        _repack_body,
        grid=(ng,),
        in_specs=[pl.BlockSpec((D, G), lambda j: (0, j))],
        out_specs=pl.BlockSpec((H, W), lambda j: (j, 0)),
        out_shape=jax.ShapeDtypeStruct((ng * H, W), jnp.float32),
    )(tab_t)


CH = 256


def _remap(idx_v):
    for k in range(CH // 16):
        sl = pl.ds(k * 16, 16)
        v = idx_v[sl]
        idx_v[sl] = lax.bitwise_or(
            lax.shift_left(lax.shift_right_logical(v, 14), 13),
            lax.bitwise_and(v, H - 1))


def _pipe_gather(units, base, idx2, rows2, gsem2, ssem2):
    n = len(units)
    handles = [None] * n
    for k in range(n + 1):
        if k < n:
            ih, tab, out, c = units[k]
            b = k % 2
            if k >= 2:
                handles[k - 2][3].wait()
            sl = pl.ds(base + c * CH, CH)
            pltpu.sync_copy(ih.at[sl], idx2[b])
            _remap(idx2[b])
            g = pltpu.async_copy(tab.at[idx2[b]], rows2[b], gsem2[b])
            handles[k] = [g, out, sl, None, b]
        if 1 <= k <= n:
            h = handles[k - 1]
            h[0].wait()
            h[3] = pltpu.async_copy(rows2[h[4]], h[1].at[h[2]], ssem2[h[4]])
    handles[n - 1][3].wait()
    if n >= 2:
        handles[n - 2][3].wait()


def _gather3_body(item_idx, pos_idx, neg_idx, item_tab, tag_tab,
                  i_out, p_out, n_out,
                  idx_a, idx_b, rows_a, rows_b, gsem_a, gsem_b,
                  ssem_a, ssem_b):
    wid = lax.axis_index("s") * NC + lax.axis_index("c")
    base = wid * BPW
    jobs = ((item_idx, item_tab, i_out),
            (pos_idx, tag_tab, p_out),
            (neg_idx, tag_tab, n_out))
    units = [(ih, tab, out, c) for ih, tab, out in jobs
             for c in range(BPW // CH)]
    _pipe_gather(units, base, (idx_a, idx_b), (rows_a, rows_b),
                 (gsem_a, gsem_b), (ssem_a, ssem_b))


def _gather1_body(user_idx, user_tab, u_out,
                  idx_a, idx_b, rows_a, rows_b, gsem_a, gsem_b,
                  ssem_a, ssem_b):
    wid = lax.axis_index("s") * NC + lax.axis_index("c")
    base = wid * BPW
    units = [(user_idx, user_tab, u_out, c) for c in range(BPW // CH)]
    _pipe_gather(units, base, (idx_a, idx_b), (rows_a, rows_b),
                 (gsem_a, gsem_b), (ssem_a, ssem_b))


def _sc_mesh():
    return plsc.VectorSubcoreMesh(core_axis_name="c", subcore_axis_name="s",
                                  num_cores=NC, num_subcores=NS)


_SC_SCRATCH = lambda: [
    pltpu.VMEM((CH,), jnp.int32),
    pltpu.VMEM((CH,), jnp.int32),
    pltpu.VMEM((CH, W), jnp.float32),
    pltpu.VMEM((CH, W), jnp.float32),
    pltpu.SemaphoreType.DMA,
    pltpu.SemaphoreType.DMA,
    pltpu.SemaphoreType.DMA,
    pltpu.SemaphoreType.DMA,
]


def _gather3(item, pos_tag, neg_tag, item_tab2, tag_tab2):
    emb = jax.ShapeDtypeStruct((B, W), jnp.float32)
    run = pl.kernel(
        _gather3_body,
        out_type=(emb, emb, emb),
        mesh=_sc_mesh(),
        scratch_types=_SC_SCRATCH(),
    )
    return run(item, pos_tag, neg_tag, item_tab2, tag_tab2)


def _gather1(user, user_tab2):
    emb = jax.ShapeDtypeStruct((B, W), jnp.float32)
    run = pl.kernel(
        _gather1_body,
        out_type=emb,
        mesh=_sc_mesh(),
        scratch_types=_SC_SCRATCH(),
    )
    return run(user, user_tab2)


def _sel(x2, idx):
    par = lax.bitwise_and(lax.shift_right_logical(idx, 13), 1) == 1
    return jnp.where(par, x2[:, D:], x2[:, :D])


def _score_body(u_ref, i_ref, p_ref, n_ref, ui_ref, ii_ref, pi_ref, ni_ref,
                e_ref, c_ref, out_ref):
    u_emb = _sel(u_ref[...], ui_ref[...])
    i_emb = _sel(i_ref[...], ii_ref[...])
    u_bf = u_emb.astype(jnp.bfloat16)
    i_bf = i_emb.astype(jnp.bfloat16)
    u_exp = jnp.dot(u_bf, e_ref[...],
                    preferred_element_type=jnp.float32).astype(jnp.bfloat16)
    i_tiled = pltpu.repeat(i_bf, D, axis=1)
    p_outer = u_exp * i_tiled
    w = jnp.dot(p_outer, c_ref[...], preferred_element_type=jnp.float32)
    d = _sel(p_ref[...], pi_ref[...]) - _sel(n_ref[...], ni_ref[...])
    s = jnp.sum(w * d, axis=1)
    ls = jnp.minimum(s, 0.0) - jnp.log(1.0 + jnp.exp(-jnp.abs(s)))
    part = jnp.sum(ls)

    @pl.when(pl.program_id(0) == 0)
    def _init():
        out_ref[0, 0] = 0.0

    out_ref[0, 0] += part

    @pl.when(pl.program_id(0) == pl.num_programs(0) - 1)
    def _fin():
        out_ref[0, 0] = out_ref[0, 0] * (-1.0 / B)


def _score(u2, i2, p2, n2, ui, ii, pi, ni, core_tensor):
    e_mat = jnp.repeat(jnp.eye(D, dtype=jnp.bfloat16), D, axis=1)
    c_mat = core_tensor.reshape(D * D, D).astype(jnp.bfloat16)
    row = pl.BlockSpec((BK, W), lambda i: (i, 0))
    col = pl.BlockSpec((BK, 1), lambda i: (i, 0))
    loss = pl.pallas_call(
        _score_body,
        grid=(B // BK,),
        in_specs=[
            row, row, row, row,
            col, col, col, col,
            pl.BlockSpec((D, D * D), lambda i: (0, 0)),
            pl.BlockSpec((D * D, D), lambda i: (0, 0)),
        ],
        out_specs=pl.BlockSpec((1, 1), lambda i: (0, 0),
                               memory_space=pltpu.SMEM),
        out_shape=jax.ShapeDtypeStruct((1, 1), jnp.float32),
    )(u2, i2, p2, n2, ui, ii, pi, ni, e_mat, c_mat)
    return loss[0, 0]


def kernel(user, item, pos_tag, neg_tag, user_table, item_table,
           good_tag_table, core_tensor):
    it2 = _repack(item_table.T)
    tt2 = _repack(good_tag_table.T)
    i2, p2, n2 = _gather3(item, pos_tag, neg_tag, it2, tt2)
    ut2 = _repack(user_table.T)
    u2 = _gather1(user, ut2)
    col = lambda idx: idx.reshape(B, 1)
    return _score(u2, i2, p2, n2, col(user), col(item), col(pos_tag),
                  col(neg_tag), core_tensor)
